# Initial kernel scaffold; baseline (speedup 1.0000x reference)
import jax
import jax.numpy as jnp
from jax.experimental import pallas as pl


def kernel(pred, target):
    raise NotImplementedError("write your pallas kernel here")


# speedup vs baseline: 3.2869x; 3.2869x over previous
import jax
import jax.numpy as jnp
from jax import lax
from jax.experimental import pallas as pl


def _roll(x, sh, axis):
    if axis == 0:
        return jnp.concatenate([x[-sh:, :], x[:-sh, :]], axis=0)
    return jnp.concatenate([x[:, -sh:], x[:, :-sh]], axis=1)


def _lovasz_body(p_ref, t_ref, o_ref):
    p = p_ref[0]
    t = t_ref[0]
    R, C = p.shape
    N = R * C
    LOGN = N.bit_length() - 1

    row = lax.broadcasted_iota(jnp.int32, (R, C), 0)
    lane = lax.broadcasted_iota(jnp.int32, (R, C), 1)

    signs = 2.0 * t - 1.0
    e = 1.0 - p * signs
    bits = lax.bitcast_convert_type(e, jnp.int32)
    bits = (bits & jnp.int32(-2)) | t.astype(jnp.int32)
    y = bits ^ ((bits >> 31) & jnp.int32(0x7FFFFFFF))
    z = ~y

    def bit0(s):
        if s >= C:
            return (row & (s // C)) == 0
        return (lane & s) == 0

    for k in range(1, LOGN + 1):
        bk = bit0(1 << k) if (1 << k) < N else jnp.full((R, C), True)
        for j in range(k - 1, -1, -1):
            s = 1 << j
            bs = bit0(s)
            if s >= C:
                u = s // C
                up = _roll(z, R - u, 0)
                dn = _roll(z, u, 0)
            else:
                up = _roll(z, C - s, 1)
                dn = _roll(z, s, 1)
            zp = jnp.where(bs, up, dn)
            take_min = bs == bk
            z = jnp.where(take_min, jnp.minimum(z, zp), jnp.maximum(z, zp))

    y_s = ~z
    bits_s = y_s ^ ((y_s >> 31) & jnp.int32(0x7FFFFFFF))
    t_s = (bits_s & 1).astype(jnp.float32)
    e_s = lax.bitcast_convert_type(bits_s, jnp.float32)

    ia = lax.broadcasted_iota(jnp.int32, (C, C), 0)
    ib = lax.broadcasted_iota(jnp.int32, (C, C), 1)
    tri = (ia <= ib).astype(jnp.float32)
    lanecum = jnp.dot(t_s, tri, preferred_element_type=jnp.float32)
    rowsum = lanecum[:, C - 1:C]
    acc = rowsum
    sh = 1
    while sh < R:
        shifted = jnp.concatenate([jnp.zeros((sh, 1), jnp.float32), acc[:-sh, :]], axis=0)
        acc = acc + shifted
        sh *= 2
    cum_t = lanecum + (acc - rowsum)

    gts = jnp.sum(t_s)
    cnt = (row * C + lane + 1).astype(jnp.float32)
    cum1 = cnt - cum_t
    inter = gts - cum_t
    union = gts + cum1
    jacc = 1.0 - inter / jnp.maximum(union, 1e-6)
    wrapped = _roll(jacc, 1, 1)
    lastcol_dn = jnp.concatenate(
        [jnp.zeros((1, 1), jnp.float32), jacc[:-1, C - 1:C]], axis=0)
    prev = jnp.where(lane == 0, lastcol_dn, wrapped)
    grad = jacc - prev
    loss = jnp.sum(jnp.maximum(e_s, 0.0) * grad)
    o_ref[0, 0, :] = jnp.broadcast_to(loss, (C,))


def _run(pred, target, interpret=False):
    B = pred.shape[0]
    C = 128
    R = (pred.shape[1] * pred.shape[2]) // C
    p = pred.reshape(B, R, C)
    t = target.reshape(B, R, C)
    losses = pl.pallas_call(
        _lovasz_body,
        grid=(B,),
        in_specs=[
            pl.BlockSpec((1, R, C), lambda i: (i, 0, 0)),
            pl.BlockSpec((1, R, C), lambda i: (i, 0, 0)),
        ],
        out_specs=pl.BlockSpec((1, 1, C), lambda i: (i, 0, 0)),
        out_shape=jax.ShapeDtypeStruct((B, 1, C), jnp.float32),
        interpret=interpret,
    )(p, t)
    total = jnp.sum(losses[:, 0, 0]) / B
    return jnp.where(jnp.isfinite(total), total, jnp.zeros((), jnp.float32))


def kernel(pred, target):
    return _run(pred, target)
